# Initial kernel scaffold; baseline (speedup 1.0000x reference)
import jax
import jax.numpy as jnp
from jax.experimental import pallas as pl


def kernel(x, conv1_w, conv1_b, bn1_g, bn1_b, conv2_w, conv2_b, bn2_g, bn2_b, proj_w, proj_b, wq, bq, wk, bk, wv, bv, wo, bo, pe, idx_sample):
    raise NotImplementedError("write your pallas kernel here")


# speedup vs baseline: 1.5895x; 1.5895x over previous
import math

import jax
import jax.numpy as jnp
from jax.experimental import pallas as pl
from jax.experimental.pallas import tpu as pltpu

_B, _L, _IN, _D, _H, _DH = 4, 512, 32, 1024, 16, 64
_U = 35
_NTOP = 35
_NEG = -3.0e38

_PREC = jax.lax.Precision.HIGHEST


def _gelu(x):
    return x * 0.5 * (1.0 + jax.lax.erf(x * (1.0 / math.sqrt(2.0))))


def _dot(a, b):
    return jax.lax.dot_general(a, b, (((1,), (0,)), ((), ())),
                               preferred_element_type=jnp.float32,
                               precision=_PREC)


def _counts_body(idx_ref, c_ref):
    lane = jax.lax.broadcasted_iota(jnp.int32, (_L, _L), 1)
    acc = jnp.zeros((_L, _L), jnp.float32)
    for u in range(_U):
        col = idx_ref[:, u : u + 1]
        acc = acc + jnp.where(col == lane, 1.0, 0.0)
    c_ref[:, :] = acc


def _conv_body(xc_ref, w1_ref, c1_ref, w2a_ref, w2b_ref, w2c_ref, c2_ref,
               h2_ref, h1_ref):
    xc = xc_ref[0]
    h1_ref[:, :] = _gelu(_dot(xc, w1_ref[:]) + c1_ref[:])
    z = jnp.zeros((1, _D), jnp.float32)
    h1 = h1_ref[:, :]
    h1m = jnp.concatenate([z, h1[:-1]], axis=0)
    h1p = jnp.concatenate([h1[1:], z], axis=0)
    h2_ref[0] = _gelu(_dot(h1m, w2a_ref[:]) + _dot(h1, w2b_ref[:])
                      + _dot(h1p, w2c_ref[:]) + c2_ref[:])


def _qkv_body(h2_ref, pw_ref, pb_ref, wq_ref, bq_ref, wk_ref, bk_ref,
              wv_ref, bv_ref, q_ref, k_ref, v_ref, f_ref):
    f_ref[:, :] = _dot(h2_ref[0], pw_ref[:]) + pb_ref[:]
    feat = f_ref[:, :]
    q_ref[0] = _dot(feat, wq_ref[:]) + bq_ref[:]
    k_ref[0] = _dot(feat, wk_ref[:]) + bk_ref[:]
    v_ref[0] = _dot(feat, wv_ref[:]) + bv_ref[:]


def _attn_body(q_ref, k_ref, v_ref, c_ref, wo_ref, bo_ref, out_ref, ctx_ref):
    C = c_ref[:]
    mask = C > 0
    sub_i = jax.lax.broadcasted_iota(jnp.int32, (_L, _L), 0)
    lane_i = jax.lax.broadcasted_iota(jnp.int32, (_L, _L), 1)
    ident = jnp.where(sub_i == lane_i, 1.0, 0.0)

    for h in range(_H):
        sl = slice(h * _DH, (h + 1) * _DH)
        qh = q_ref[0, :, sl]
        kh = k_ref[0, :, sl]
        vh = v_ref[0, :, sl]
        S = jax.lax.dot_general(qh, kh, (((1,), (1,)), ((), ())),
                                preferred_element_type=jnp.float32,
                                precision=_PREC)
        m_max = jnp.max(jnp.where(mask, S, _NEG), axis=1, keepdims=True)
        m_sum = jnp.sum(S * C, axis=1, keepdims=True)
        M = m_max - m_sum * (1.0 / _L)
        m_row = jax.lax.dot_general(M, ident, (((0,), (0,)), ((), ())),
                                    preferred_element_type=jnp.float32,
                                    precision=_PREC)
        beats = (m_row > M) | ((m_row == M) & (lane_i < sub_i))
        rank = jnp.sum(jnp.where(beats, 1.0, 0.0), axis=1, keepdims=True)
        sel = rank < float(_NTOP)

        s_max = jnp.max(S, axis=1, keepdims=True)
        P = jnp.exp(S - s_max)
        denom = jnp.sum(P, axis=1, keepdims=True)
        upd = jax.lax.dot_general(P, vh, (((1,), (0,)), ((), ())),
                                  preferred_element_type=jnp.float32,
                                  precision=_PREC) / denom
        v_mean = jnp.sum(vh, axis=0, keepdims=True) * (1.0 / _L)
        ctx_ref[:, sl] = jnp.where(sel, upd, jnp.broadcast_to(v_mean, (_L, _DH)))

    out_ref[0] = _dot(ctx_ref[:, :], wo_ref[:]) + bo_ref[:]


def _full(shape):
    nd = len(shape)
    return pl.BlockSpec(shape, lambda b: (0,) * nd)


def _bblock(shape):
    nd = len(shape)
    return pl.BlockSpec((1,) + shape[1:], lambda b: (b,) + (0,) * (nd - 1))


def kernel(x, conv1_w, conv1_b, bn1_g, bn1_b, conv2_w, conv2_b, bn2_g, bn2_b,
           proj_w, proj_b, wq, bq, wk, bk, wv, bv, wo, bo, pe, idx_sample):
    f32 = jnp.float32
    inv = f32(1.0 / math.sqrt(1.0 + 1e-05))
    a1 = bn1_g * inv
    a2 = bn2_g * inv

    w1m = jnp.concatenate([conv1_w[:, :, t].T for t in range(3)], axis=0) * a1[None, :]
    c1 = (conv1_b * a1 + bn1_b)[None, :]
    w2a = conv2_w[:, :, 0].T * a2[None, :]
    w2b = conv2_w[:, :, 1].T * a2[None, :]
    w2c = conv2_w[:, :, 2].T * a2[None, :]
    c2 = (conv2_b * a2 + bn2_b)[None, :]

    pw = proj_w.T
    pb = pe[0, :_L, :] + proj_b[None, :]
    wqT, wkT, wvT, woT = wq.T, wk.T, wv.T, wo.T
    bq2, bk2, bv2, bo2 = bq[None, :], bk[None, :], bv[None, :], bo[None, :]

    xp = jnp.pad(x, ((0, 0), (1, 1), (0, 0)))
    xc = jnp.concatenate([xp[:, :_L, :], xp[:, 1:_L + 1, :], xp[:, 2:_L + 2, :]],
                         axis=2)

    idx = idx_sample.astype(jnp.int32)
    counts = pl.pallas_call(
        _counts_body,
        out_shape=jax.ShapeDtypeStruct((_L, _L), f32),
        in_specs=[pl.BlockSpec((_L, _U), lambda: (0, 0))],
        out_specs=pl.BlockSpec((_L, _L), lambda: (0, 0)),
    )(idx)

    h2 = pl.pallas_call(
        _conv_body,
        grid=(_B,),
        out_shape=jax.ShapeDtypeStruct((_B, _L, _D), f32),
        in_specs=[
            _bblock((_B, _L, 3 * _IN)),
            _full((3 * _IN, _D)),
            _full((1, _D)),
            _full((_D, _D)),
            _full((_D, _D)),
            _full((_D, _D)),
            _full((1, _D)),
        ],
        out_specs=_bblock((_B, _L, _D)),
        scratch_shapes=[pltpu.VMEM((_L, _D), f32)],
    )(xc, w1m, c1, w2a, w2b, w2c, c2)

    q, k, v = pl.pallas_call(
        _qkv_body,
        grid=(_B,),
        out_shape=[jax.ShapeDtypeStruct((_B, _L, _D), f32)] * 3,
        in_specs=[
            _bblock((_B, _L, _D)),
            _full((_D, _D)),
            _full((_L, _D)),
            _full((_D, _D)),
            _full((1, _D)),
            _full((_D, _D)),
            _full((1, _D)),
            _full((_D, _D)),
            _full((1, _D)),
        ],
        out_specs=[_bblock((_B, _L, _D))] * 3,
        scratch_shapes=[pltpu.VMEM((_L, _D), f32)],
    )(h2, pw, pb, wqT, bq2, wkT, bk2, wvT, bv2)

    out = pl.pallas_call(
        _attn_body,
        grid=(_B,),
        out_shape=jax.ShapeDtypeStruct((_B, _L, _D), f32),
        in_specs=[
            _bblock((_B, _L, _D)),
            _bblock((_B, _L, _D)),
            _bblock((_B, _L, _D)),
            _full((_L, _L)),
            _full((_D, _D)),
            _full((1, _D)),
        ],
        out_specs=_bblock((_B, _L, _D)),
        scratch_shapes=[pltpu.VMEM((_L, _D), f32)],
    )(q, k, v, counts, woT, bo2)
    return out
